# baseline (speedup 1.0000x reference)
import functools

import jax
import jax.numpy as jnp
from jax import lax
from jax.experimental import pallas as pl
from jax.experimental.pallas import tpu as pltpu
from jax.experimental.pallas import tpu_sc as plsc

EMB = 64
LANE = 128
K = 8
NW = 32
NC = 2


def _emb_body(idx_hbm, table_hbm, out_hbm, idx_v, rows_v, sem):
    wid = lax.axis_index("s") * NC + lax.axis_index("c")
    rows_total = idx_hbm.shape[0]
    per_w = rows_total // NW
    groups = per_w // K
    base = wid * per_w

    def body(g, carry):
        r0 = base + g * K
        pltpu.sync_copy(idx_hbm.at[pl.ds(r0, K)], idx_v)
        copies = [
            pltpu.async_copy(table_hbm.at[idx_v.at[j]], rows_v.at[j], sem)
            for j in range(K)
        ]
        for c in copies:
            c.wait()
        pltpu.sync_copy(rows_v, out_hbm.at[pl.ds(r0, K)])
        return carry

    lax.fori_loop(0, groups, body, 0)


def kernel(idx_seqs, emb_weight):
    B, S = idx_seqs.shape
    total = B * S
    assert total % (LANE * NW * K) == 0
    n_rows = total // LANE

    flat_idx = idx_seqs.astype(jnp.int32).reshape(n_rows, LANE)

    run = functools.partial(
        pl.kernel,
        out_type=jax.ShapeDtypeStruct((n_rows, LANE, EMB), jnp.float32),
        mesh=plsc.VectorSubcoreMesh(core_axis_name="c", subcore_axis_name="s"),
        scratch_types=[
            pltpu.VMEM((K, LANE), jnp.int32),
            pltpu.VMEM((K, LANE, EMB), jnp.float32),
            pltpu.SemaphoreType.DMA,
        ],
        compiler_params=pltpu.CompilerParams(use_tc_tiling_on_sc=False),
    )(_emb_body)

    out = run(flat_idx, emb_weight)
    return out.reshape(B, S, EMB)


# speedup vs baseline: 1.0176x; 1.0176x over previous
import functools

import jax
import jax.numpy as jnp
from jax import lax
from jax.experimental import pallas as pl
from jax.experimental.pallas import tpu as pltpu
from jax.experimental.pallas import tpu_sc as plsc

EMB = 64
LANE = 128
NBUF = 8
H = NBUF // 2
NW = 32
NC = 2


def _emb_body(idx_hbm, table_hbm, out_hbm, idx_all, rows, sem_g, sem_s):
    wid = lax.axis_index("s") * NC + lax.axis_index("c")
    rows_total = idx_hbm.shape[0]
    per_w = rows_total // NW
    steps = per_w // NBUF
    base = wid * per_w

    pltpu.sync_copy(idx_hbm.at[pl.ds(base, per_w)], idx_all)

    def gather_issue(b, g):
        pltpu.async_copy(table_hbm.at[idx_all.at[g]], rows.at[b], sem_g.at[b])

    def gather_wait(b, g):
        pltpu.make_async_copy(
            table_hbm.at[idx_all.at[g]], rows.at[b], sem_g.at[b]
        ).wait()

    def store_issue(b, g):
        pltpu.async_copy(rows.at[b], out_hbm.at[base + g], sem_s.at[b])

    def store_wait(b, g):
        pltpu.make_async_copy(
            rows.at[b], out_hbm.at[base + g], sem_s.at[b]
        ).wait()

    def outer(t, carry):
        for b in range(NBUF):
            g = t * NBUF + b
            b2 = (b + NBUF - H) % NBUF
            gs = g - H
            if b < H:
                @pl.when(t >= 1)
                def _():
                    store_wait(b, g - NBUF)
                    gather_issue(b, g)
                    gather_wait(b2, gs)
                    store_issue(b2, gs)

                @pl.when(t < 1)
                def _():
                    gather_issue(b, g)
            else:
                @pl.when(t >= 1)
                def _():
                    store_wait(b, g - NBUF)

                gather_issue(b, g)
                gather_wait(b2, gs)
                store_issue(b2, gs)
        return carry

    lax.fori_loop(0, steps, outer, 0)

    for k in range(H):
        g = per_w - H + k
        b2 = g % NBUF
        gather_wait(b2, g)
        store_issue(b2, g)
    for b in range(NBUF):
        g = per_w - NBUF + b
        store_wait(b, g)


def kernel(idx_seqs, emb_weight):
    B, S = idx_seqs.shape
    total = B * S
    assert total % (LANE * NW * NBUF) == 0
    n_rows = total // LANE

    flat_idx = idx_seqs.astype(jnp.int32).reshape(n_rows, LANE)

    run = functools.partial(
        pl.kernel,
        out_type=jax.ShapeDtypeStruct((n_rows, LANE, EMB), jnp.float32),
        mesh=plsc.VectorSubcoreMesh(core_axis_name="c", subcore_axis_name="s"),
        scratch_types=[
            pltpu.VMEM((n_rows // NW, LANE), jnp.int32),
            pltpu.VMEM((NBUF, LANE, EMB), jnp.float32),
            pltpu.SemaphoreType.DMA((NBUF,)),
            pltpu.SemaphoreType.DMA((NBUF,)),
        ],
        compiler_params=pltpu.CompilerParams(use_tc_tiling_on_sc=False),
    )(_emb_body)

    out = run(flat_idx, emb_weight)
    return out.reshape(B, S, EMB)


# speedup vs baseline: 1.2431x; 1.2216x over previous
import functools

import jax
import jax.numpy as jnp
from jax import lax
from jax.experimental import pallas as pl
from jax.experimental.pallas import tpu as pltpu
from jax.experimental.pallas import tpu_sc as plsc

EMB = 64
PADW = 128
LANE = 128
NBUF = 5
H = 2
NW = 32
NC = 2


def _emb_body(idx_hbm, table_hbm, out_hbm, idx_all, rows, sem_g, sem_s):
    wid = lax.axis_index("s") * NC + lax.axis_index("c")
    rows_total = idx_hbm.shape[0]
    per_w = rows_total // NW
    steps = per_w // NBUF
    base = wid * per_w

    pltpu.sync_copy(idx_hbm.at[pl.ds(base, per_w)], idx_all)

    def gather_issue(b, g):
        pltpu.async_copy(table_hbm.at[idx_all.at[g]], rows.at[b], sem_g.at[b])

    def gather_wait(b, g):
        pltpu.make_async_copy(
            table_hbm.at[idx_all.at[g]], rows.at[b], sem_g.at[b]
        ).wait()

    def store_issue(b, g):
        pltpu.async_copy(
            rows.at[b], out_hbm.at[pl.ds((base + g) * LANE, LANE)], sem_s.at[b]
        )

    def store_wait(b, g):
        pltpu.make_async_copy(
            rows.at[b], out_hbm.at[pl.ds((base + g) * LANE, LANE)], sem_s.at[b]
        ).wait()

    def outer(t, carry):
        for b in range(NBUF):
            g = t * NBUF + b
            b2 = (b + NBUF - H) % NBUF
            gs = g - H
            if b < H:
                @pl.when(t >= 1)
                def _():
                    store_wait(b, g - NBUF)
                    gather_issue(b, g)
                    gather_wait(b2, gs)
                    store_issue(b2, gs)

                @pl.when(t < 1)
                def _():
                    gather_issue(b, g)
            else:
                @pl.when(t >= 1)
                def _():
                    store_wait(b, g - NBUF)

                gather_issue(b, g)
                gather_wait(b2, gs)
                store_issue(b2, gs)
        return carry

    lax.fori_loop(0, steps, outer, 0)

    for k in range(H):
        g = per_w - H + k
        b2 = g % NBUF
        gather_wait(b2, g)
        store_issue(b2, g)
    for b in range(NBUF):
        g = per_w - NBUF + b
        store_wait(b, g)


def kernel(idx_seqs, emb_weight):
    B, S = idx_seqs.shape
    total = B * S
    assert total % (LANE * NW * NBUF) == 0
    n_rows = total // LANE

    flat_idx = idx_seqs.astype(jnp.int32).reshape(n_rows, LANE)
    table_pad = jnp.pad(emb_weight, ((0, 0), (0, PADW - EMB)))

    run = functools.partial(
        pl.kernel,
        out_type=jax.ShapeDtypeStruct((total, PADW), jnp.float32),
        mesh=plsc.VectorSubcoreMesh(core_axis_name="c", subcore_axis_name="s"),
        scratch_types=[
            pltpu.VMEM((n_rows // NW, LANE), jnp.int32),
            pltpu.VMEM((NBUF, LANE, PADW), jnp.float32),
            pltpu.SemaphoreType.DMA((NBUF,)),
            pltpu.SemaphoreType.DMA((NBUF,)),
        ],
        compiler_params=pltpu.CompilerParams(use_tc_tiling_on_sc=True),
    )(_emb_body)

    out = run(flat_idx, table_pad)
    return out[:, :EMB].reshape(B, S, EMB)
